# baseline (speedup 1.0000x reference)
import functools

import jax
import jax.numpy as jnp
from jax import lax
from jax.experimental import pallas as pl
from jax.experimental.pallas import tpu as pltpu
from jax.experimental.pallas import tpu_sc as plsc

_STRIDES = (8, 16, 32, 64, 128)
_HWS = (4096, 1024, 256, 64, 16)
_NIMG = 8
_NCLS = 80
_THRESH = 0.05
_NTOT = 436480
_NSUB = 16
_CHUNK = _NTOT // 32 * 2
_NB = 4096
_BSHIFT = 19
_CAP = 64


def _score_body(*refs):
    lg_refs = refs[0:5]
    ct_refs = refs[5:10]
    out_refs = refs[10:15]
    for lg_ref, ct_ref, o_ref in zip(lg_refs, ct_refs, out_refs):
        lg = jax.nn.sigmoid(lg_ref[...])
        ct = jax.nn.sigmoid(ct_ref[...])
        o_ref[...] = jnp.where(lg > _THRESH, lg * ct, 0.0)


def _dense_scores(logits, ctrs):
    in_specs = (
        [pl.BlockSpec((1, _NCLS, hw), lambda i: (i, 0, 0)) for hw in _HWS]
        + [pl.BlockSpec((1, 1, hw), lambda i: (i, 0, 0)) for hw in _HWS]
    )
    out_specs = [pl.BlockSpec((1, _NCLS, hw), lambda i: (i, 0, 0)) for hw in _HWS]
    out_shape = [jax.ShapeDtypeStruct((_NIMG, _NCLS, hw), jnp.float32) for hw in _HWS]
    return pl.pallas_call(
        _score_body,
        grid=(_NIMG,),
        in_specs=in_specs,
        out_specs=out_specs,
        out_shape=out_shape,
    )(*logits, *ctrs)


def _hist_body(keys_hbm, hist_hbm, buf, hist_v):
    cid = lax.axis_index("c")
    sid = lax.axis_index("s")
    zeros16 = jnp.zeros((16,), jnp.int32)
    ones16 = jnp.ones((16,), jnp.int32)

    def _z(i, _):
        hist_v[pl.ds(i * 16, 16)] = zeros16
        return 0
    lax.fori_loop(0, 4 * _NB // 16, _z, 0)

    for im in range(4):
        img = cid * 4 + im
        pltpu.sync_copy(
            keys_hbm.at[pl.ds(img * _NTOT + sid * _CHUNK, _CHUNK)], buf)

        def _acc(i, _, im=im):
            k = buf[pl.ds(i * 16, 16)]
            b = lax.shift_right_logical(k, _BSHIFT) + im * _NB
            plsc.addupdate_scatter(hist_v, [b], ones16)
            return 0
        lax.fori_loop(0, _CHUNK // 16, _acc, 0)

    wid = cid * _NSUB + sid
    pltpu.sync_copy(hist_v, hist_hbm.at[pl.ds(wid * 4 * _NB, 4 * _NB)])


def _sc_histogram(keys):
    mesh = plsc.VectorSubcoreMesh(core_axis_name="c", subcore_axis_name="s")
    f = functools.partial(
        pl.kernel,
        out_type=jax.ShapeDtypeStruct((2 * _NSUB * 4 * _NB,), jnp.int32),
        mesh=mesh,
        compiler_params=pltpu.CompilerParams(needs_layout_passes=False),
        scratch_types=[
            pltpu.VMEM((_CHUNK,), jnp.int32),
            pltpu.VMEM((4 * _NB,), jnp.int32),
        ],
    )(_hist_body)
    part = f(keys.reshape(-1))
    return part.reshape(2, _NSUB, 4, _NB).sum(axis=1).reshape(_NIMG, _NB)


def kernel(logits0, logits1, logits2, logits3, logits4,
           reg0, reg1, reg2, reg3, reg4,
           ctr0, ctr1, ctr2, ctr3, ctr4,
           loc0, loc1, loc2, loc3, loc4,
           image_sizes):
    logits = [logits0, logits1, logits2, logits3, logits4]
    regs = [reg0, reg1, reg2, reg3, reg4]
    ctrs = [ctr0, ctr1, ctr2, ctr3, ctr4]
    locs = [loc0, loc1, loc2, loc3, loc4]

    lg3 = [l.reshape(_NIMG, _NCLS, hw) for l, hw in zip(logits, _HWS)]
    ct3 = [c.reshape(_NIMG, 1, hw) for c, hw in zip(ctrs, _HWS)]
    scores = _dense_scores(lg3, ct3)

    keys = lax.bitcast_convert_type(
        jnp.concatenate([s.reshape(_NIMG, -1) for s in scores], axis=1),
        jnp.int32)

    hist = _sc_histogram(keys)

    revc = jnp.cumsum(hist[:, ::-1], axis=1)[:, ::-1]
    bidx = jnp.arange(_NB, dtype=jnp.int32)
    bstar = jnp.max(jnp.where(revc >= 256, bidx[None, :], 0), axis=1)
    tkey = bstar << _BSHIFT

    masked = jnp.where(keys >= tkey[:, None], keys, 0)
    top_k_keys, top_i = jax.lax.top_k(masked, 256)
    top_s = lax.bitcast_convert_type(top_k_keys, jnp.float32)

    offs = [0]
    for hw in _HWS:
        offs.append(offs[-1] + _NCLS * hw)
    offs_arr = jnp.array(offs[:5], dtype=jnp.int32)
    lvl = jnp.sum(top_i[:, :, None] >= offs_arr[None, None, :],
                  axis=-1).astype(jnp.int32) - 1
    local = top_i - offs_arr[lvl]
    hw_arr = jnp.array(_HWS, dtype=jnp.int32)
    cls = (local // hw_arr[lvl]).astype(jnp.int32)
    pos = local % hw_arr[lvl]
    posoff = jnp.array([0, 4096, 5120, 5376, 5440], dtype=jnp.int32)
    gpos = posoff[lvl] + pos

    loc_all = jnp.concatenate(locs, axis=0)
    rg_all = jnp.concatenate(
        [jnp.transpose((r * s).reshape(_NIMG, 4, hw), (0, 2, 1))
         for r, s, hw in zip(regs, _STRIDES, _HWS)],
        axis=1)

    per_loc = loc_all[gpos]
    per_reg = jnp.take_along_axis(rg_all, gpos[:, :, None], axis=1)

    x1 = per_loc[:, :, 0] - per_reg[:, :, 0]
    y1 = per_loc[:, :, 1] - per_reg[:, :, 1]
    x2 = per_loc[:, :, 0] + per_reg[:, :, 2]
    y2 = per_loc[:, :, 1] + per_reg[:, :, 3]
    fb = jnp.stack([x1, y1, x2, y2], axis=2)

    fs = jnp.sqrt(jnp.maximum(top_s, 0.0)) * (top_s > 0)
    return fb, fs, cls, lvl


# speedup vs baseline: 9.3651x; 9.3651x over previous
import functools

import jax
import jax.numpy as jnp
from jax import lax
from jax.experimental import pallas as pl
from jax.experimental.pallas import tpu as pltpu
from jax.experimental.pallas import tpu_sc as plsc

_STRIDES = (8, 16, 32, 64, 128)
_HWS = (4096, 1024, 256, 64, 16)
_NIMG = 8
_NCLS = 80
_THRESH = 0.05
_NTOT = 436480
_NSUB = 16
_CHUNK = _NTOT // 32 * 2
_NB = 4096
_BSHIFT = 19
_CAP = 64


def _score_body(*refs):
    lg_refs = refs[0:5]
    ct_refs = refs[5:10]
    out_refs = refs[10:15]
    for lg_ref, ct_ref, o_ref in zip(lg_refs, ct_refs, out_refs):
        lg = jax.nn.sigmoid(lg_ref[...])
        ct = jax.nn.sigmoid(ct_ref[...])
        o_ref[...] = jnp.where(lg > _THRESH, lg * ct, 0.0)


def _dense_scores(logits, ctrs):
    in_specs = (
        [pl.BlockSpec((1, _NCLS, hw), lambda i: (i, 0, 0)) for hw in _HWS]
        + [pl.BlockSpec((1, 1, hw), lambda i: (i, 0, 0)) for hw in _HWS]
    )
    out_specs = [pl.BlockSpec((1, _NCLS, hw), lambda i: (i, 0, 0)) for hw in _HWS]
    out_shape = [jax.ShapeDtypeStruct((_NIMG, _NCLS, hw), jnp.float32) for hw in _HWS]
    return pl.pallas_call(
        _score_body,
        grid=(_NIMG,),
        in_specs=in_specs,
        out_specs=out_specs,
        out_shape=out_shape,
    )(*logits, *ctrs)


def _hist_body(keys_hbm, hist_hbm, buf, hist_v):
    cid = lax.axis_index("c")
    sid = lax.axis_index("s")
    zeros16 = jnp.zeros((16,), jnp.int32)
    ones16 = jnp.ones((16,), jnp.int32)

    def _z(i, _):
        hist_v[pl.ds(i * 16, 16)] = zeros16
        return 0
    lax.fori_loop(0, 4 * _NB // 16, _z, 0)

    for im in range(4):
        img = cid * 4 + im
        pltpu.sync_copy(
            keys_hbm.at[pl.ds(img * _NTOT + sid * _CHUNK, _CHUNK)], buf)

        def _acc(i, _, im=im):
            k = buf[pl.ds(i * 16, 16)]
            b = lax.shift_right_logical(k, _BSHIFT) + im * _NB
            plsc.addupdate_scatter(hist_v, [b], ones16)
            return 0
        lax.fori_loop(0, _CHUNK // 16, _acc, 0)

    wid = cid * _NSUB + sid
    pltpu.sync_copy(hist_v, hist_hbm.at[pl.ds(wid * 4 * _NB, 4 * _NB)])


def _sc_histogram(keys):
    mesh = plsc.VectorSubcoreMesh(core_axis_name="c", subcore_axis_name="s")
    f = functools.partial(
        pl.kernel,
        out_type=jax.ShapeDtypeStruct((2 * _NSUB * 4 * _NB,), jnp.int32),
        mesh=mesh,
        compiler_params=pltpu.CompilerParams(needs_layout_passes=False),
        scratch_types=[
            pltpu.VMEM((_CHUNK,), jnp.int32),
            pltpu.VMEM((4 * _NB,), jnp.int32),
        ],
    )(_hist_body)
    part = f(keys.reshape(-1))
    return part.reshape(2, _NSUB, 4, _NB).sum(axis=1).reshape(_NIMG, _NB)


def _compact_body(keys_hbm, tkey_hbm, ckeys_hbm, cidx_hbm, buf, okey, oidx, tv):
    cid = lax.axis_index("c")
    sid = lax.axis_index("s")
    zeros16 = jnp.zeros((16,), jnp.int32)
    iota16 = lax.iota(jnp.int32, 16)

    pltpu.sync_copy(tkey_hbm, tv.at[pl.ds(0, _NIMG)])

    for im in range(4):
        img = cid * 4 + im
        pltpu.sync_copy(
            keys_hbm.at[pl.ds(img * _NTOT + sid * _CHUNK, _CHUNK)], buf)
        for j in range((_CAP + 16) // 16):
            okey[pl.ds(j * 16, 16)] = zeros16
            oidx[pl.ds(j * 16, 16)] = zeros16
        tvec = tv[pl.ds(0, 16)]
        t = jnp.max(jnp.where(iota16 == img, tvec, 0))

        def _step(i, off):
            k = buf[pl.ds(i * 16, 16)]
            m = k >= t
            offc = jnp.minimum(off, _CAP)
            plsc.store_compressed(okey.at[pl.ds(offc, 16)], k, mask=m)
            gi = sid * _CHUNK + i * 16 + iota16
            plsc.store_compressed(oidx.at[pl.ds(offc, 16)], gi, mask=m)
            return off + jnp.sum(m.astype(jnp.int32))
        lax.fori_loop(0, _CHUNK // 16, _step, jnp.int32(0))

        dst = img * (_NSUB * _CAP) + sid * _CAP
        pltpu.sync_copy(okey.at[pl.ds(0, _CAP)], ckeys_hbm.at[pl.ds(dst, _CAP)])
        pltpu.sync_copy(oidx.at[pl.ds(0, _CAP)], cidx_hbm.at[pl.ds(dst, _CAP)])


_NCOMP = _NSUB * _CAP


def _sc_compact(keys, tkey):
    mesh = plsc.VectorSubcoreMesh(core_axis_name="c", subcore_axis_name="s")
    f = functools.partial(
        pl.kernel,
        out_type=(jax.ShapeDtypeStruct((_NIMG * _NCOMP,), jnp.int32),
                  jax.ShapeDtypeStruct((_NIMG * _NCOMP,), jnp.int32)),
        mesh=mesh,
        compiler_params=pltpu.CompilerParams(needs_layout_passes=False),
        scratch_types=[
            pltpu.VMEM((_CHUNK,), jnp.int32),
            pltpu.VMEM((_CAP + 16,), jnp.int32),
            pltpu.VMEM((_CAP + 16,), jnp.int32),
            pltpu.VMEM((16,), jnp.int32),
        ],
    )(_compact_body)
    return f(keys.reshape(-1), tkey)


def kernel(logits0, logits1, logits2, logits3, logits4,
           reg0, reg1, reg2, reg3, reg4,
           ctr0, ctr1, ctr2, ctr3, ctr4,
           loc0, loc1, loc2, loc3, loc4,
           image_sizes):
    logits = [logits0, logits1, logits2, logits3, logits4]
    regs = [reg0, reg1, reg2, reg3, reg4]
    ctrs = [ctr0, ctr1, ctr2, ctr3, ctr4]
    locs = [loc0, loc1, loc2, loc3, loc4]

    lg3 = [l.reshape(_NIMG, _NCLS, hw) for l, hw in zip(logits, _HWS)]
    ct3 = [c.reshape(_NIMG, 1, hw) for c, hw in zip(ctrs, _HWS)]
    scores = _dense_scores(lg3, ct3)

    keys = lax.bitcast_convert_type(
        jnp.concatenate([s.reshape(_NIMG, -1) for s in scores], axis=1),
        jnp.int32)

    hist = _sc_histogram(keys)

    revc = jnp.cumsum(hist[:, ::-1], axis=1)[:, ::-1]
    bidx = jnp.arange(_NB, dtype=jnp.int32)
    bstar = jnp.max(jnp.where(revc >= 256, bidx[None, :], 0), axis=1)
    tkey = bstar << _BSHIFT

    ckeys, cidx = _sc_compact(keys, tkey)
    ckeys = ckeys.reshape(_NIMG, _NCOMP)
    cidx = cidx.reshape(_NIMG, _NCOMP)

    top_k_keys, top_slot = jax.lax.top_k(ckeys, 256)
    top_i = jnp.take_along_axis(cidx, top_slot, axis=1)
    top_s = lax.bitcast_convert_type(top_k_keys, jnp.float32)

    offs = [0]
    for hw in _HWS:
        offs.append(offs[-1] + _NCLS * hw)
    offs_arr = jnp.array(offs[:5], dtype=jnp.int32)
    lvl = jnp.sum(top_i[:, :, None] >= offs_arr[None, None, :],
                  axis=-1).astype(jnp.int32) - 1
    local = top_i - offs_arr[lvl]
    hw_arr = jnp.array(_HWS, dtype=jnp.int32)
    cls = (local // hw_arr[lvl]).astype(jnp.int32)
    pos = local % hw_arr[lvl]
    posoff = jnp.array([0, 4096, 5120, 5376, 5440], dtype=jnp.int32)
    gpos = posoff[lvl] + pos

    loc_all = jnp.concatenate(locs, axis=0)
    rg_all = jnp.concatenate(
        [jnp.transpose((r * s).reshape(_NIMG, 4, hw), (0, 2, 1))
         for r, s, hw in zip(regs, _STRIDES, _HWS)],
        axis=1)

    per_loc = loc_all[gpos]
    per_reg = jnp.take_along_axis(rg_all, gpos[:, :, None], axis=1)

    x1 = per_loc[:, :, 0] - per_reg[:, :, 0]
    y1 = per_loc[:, :, 1] - per_reg[:, :, 1]
    x2 = per_loc[:, :, 0] + per_reg[:, :, 2]
    y2 = per_loc[:, :, 1] + per_reg[:, :, 3]
    fb = jnp.stack([x1, y1, x2, y2], axis=2)

    fs = jnp.sqrt(jnp.maximum(top_s, 0.0)) * (top_s > 0)
    return fb, fs, cls, lvl
